# baseline (speedup 1.0000x reference)
import functools
import math

import jax
import jax.numpy as jnp
from jax import lax
from jax.experimental import pallas as pl
from jax.experimental.pallas import tpu as pltpu
from jax.experimental.pallas import tpu_sc as plsc

B, L, D = 1024, 200, 256
N = B * L
NC, NS, LANES = 2, 16, 16
NW = NC * NS
PER_W = N // NW
C = 64
NCHUNK = PER_W // C
NPAIR = NCHUNK // 2
SCALE = 16.0

_mesh = plsc.VectorSubcoreMesh(core_axis_name="c", subcore_axis_name="s")


@functools.partial(
    pl.kernel,
    mesh=_mesh,
    out_type=jax.ShapeDtypeStruct((N, D), jnp.float32),
    scratch_types=[
        pltpu.VMEM((PER_W,), jnp.int32),
        pltpu.VMEM((PER_W,), jnp.int32),
        pltpu.VMEM((PER_W,), jnp.int32),
        pltpu.VMEM((C, D), jnp.float32),
        pltpu.VMEM((C, D), jnp.float32),
        pltpu.VMEM((C, D), jnp.float32),
        pltpu.VMEM((C, D), jnp.float32),
        pltpu.VMEM((C, D), jnp.float32),
        pltpu.VMEM((C, D), jnp.float32),
        pltpu.SemaphoreType.DMA,
        pltpu.SemaphoreType.DMA,
    ],
)
def _embed_sum(vt, ct, pt, val_tab, coord_tab, pos_tab, out,
               idx_v, idx_c, idx_p,
               a0, b0, c0, a1, b1, c1, sem0, sem1):
    wid = lax.axis_index("s") * NC + lax.axis_index("c")
    base = wid * PER_W
    rows = ((a0, b0, c0, sem0), (a1, b1, c1, sem1))

    pltpu.sync_copy(vt.at[pl.ds(base, PER_W)], idx_v)
    pltpu.sync_copy(ct.at[pl.ds(base, PER_W)], idx_c)
    pltpu.sync_copy(pt.at[pl.ds(base, PER_W)], idx_p)

    def descs(k, s):
        ra, rb, rc, sem = rows[s]
        o = k * C
        return (
            pltpu.make_async_copy(val_tab.at[idx_v.at[pl.ds(o, C)]], ra, sem),
            pltpu.make_async_copy(coord_tab.at[idx_c.at[pl.ds(o, C)]], rb, sem),
            pltpu.make_async_copy(pos_tab.at[idx_p.at[pl.ds(o, C)]], rc, sem),
        )

    def start(k, s):
        for d in descs(k, s):
            d.start()

    def wait(k, s):
        for d in descs(k, s):
            d.wait()

    def process(k, s):
        ra, rb, rc, _ = rows[s]

        def tok(t, carry):
            for j in range(D // LANES):
                sl = pl.ds(j * LANES, LANES)
                ra[t, sl] = (ra[t, sl] + rb[t, sl] + rc[t, sl]) * SCALE
            return carry

        lax.fori_loop(0, C, tok, 0)
        pltpu.sync_copy(ra, out.at[pl.ds(base + k * C, C)])

    start(0, 0)

    def pair(i, carry):
        k0 = 2 * i
        start(k0 + 1, 1)
        wait(k0, 0)
        process(k0, 0)

        @pl.when(i < NPAIR - 1)
        def _():
            start(k0 + 2, 0)

        wait(k0 + 1, 1)
        process(k0 + 1, 1)
        return carry

    lax.fori_loop(0, NPAIR, pair, 0)


def kernel(value_tokens, coord_type_tokens, position_tokens,
           value_table, coord_type_table, position_table):
    vt = value_tokens.reshape(N).astype(jnp.int32)
    ct = coord_type_tokens.reshape(N).astype(jnp.int32)
    pt = position_tokens.reshape(N).astype(jnp.int32)
    out = _embed_sum(vt, ct, pt, value_table, coord_type_table, position_table)
    return out.reshape(B, L, D)
